# Initial kernel scaffold; baseline (speedup 1.0000x reference)
import jax
import jax.numpy as jnp
from jax.experimental import pallas as pl


def kernel(observed_data, observed_mask, timepoints, gt_mask, t, noise, rand_vals, sample_ratios, w1, w2):
    raise NotImplementedError("write your pallas kernel here")


# speedup vs baseline: 27.8648x; 27.8648x over previous
import functools
import numpy as np
import jax
import jax.numpy as jnp
from jax import lax
from jax.experimental import pallas as pl
from jax.experimental.pallas import tpu as pltpu

_NUM_STEPS = 50
_BETA_START = 0.0001
_BETA_END = 0.5


def _alpha_bar_np():
    beta = np.linspace(_BETA_START ** 0.5, _BETA_END ** 0.5, _NUM_STEPS) ** 2
    return np.cumprod(1.0 - beta)


def _loss_kernel(sa_ref, sb_ref, ratio_ref, od_ref, om_ref, noise_ref,
                 rand_ref, w1_ref, out_ref, *, inv_b, tie_iters):
    b = pl.program_id(0)
    om_t = jnp.transpose(om_ref[0])
    od_t = jnp.transpose(od_ref[0])
    rand = rand_ref[0]
    noise = noise_ref[0]
    w1 = w1_ref[...]
    K, L = rand.shape

    rf = rand * om_t
    vb = lax.bitcast_convert_type(rf, jnp.int32)
    num_obs = jnp.sum(om_t)
    ratio = ratio_ref[0, 0, 0]
    kkf = (num_obs * ratio + 16777216.0) - 16777216.0
    kf = jnp.maximum(kkf, 1.0)

    def vbody(_, lohi):
        lo, hi = lohi
        mid = (lo + hi) // 2
        c = jnp.sum(jnp.where(vb >= mid, 1.0, 0.0))
        take = c >= kf
        return jnp.where(take, mid, lo), jnp.where(take, hi, mid)

    v, _ = lax.fori_loop(0, 30, vbody,
                         (jnp.int32(0), jnp.int32(1 << 30)))

    n_gt = jnp.sum(jnp.where(vb > v, 1.0, 0.0))
    m_eq = jnp.sum(jnp.where(vb == v, 1.0, 0.0))
    r = kf - n_gt

    idx = (lax.broadcasted_iota(jnp.int32, (K, L), 0) * L
           + lax.broadcasted_iota(jnp.int32, (K, L), 1))

    def tbody(_, lohi):
        lo, hi = lohi
        mid = (lo + hi) // 2
        c = jnp.sum(jnp.where((vb == v) & (idx <= mid), 1.0, 0.0))
        take = c >= r
        return jnp.where(take, lo, mid + 1), jnp.where(take, mid, hi)

    need_tie = (v > 0) & (r < m_eq) & (kkf > 0)
    i_thr = lax.cond(
        need_tie,
        lambda: lax.fori_loop(0, tie_iters, tbody,
                              (jnp.int32(0), jnp.int32(K * L - 1)))[0],
        lambda: jnp.where((v > 0) & (kkf > 0),
                          jnp.int32(K * L - 1), jnp.int32(-1)))

    v_eff = jnp.where(kkf > 0, v, jnp.int32(0x7F000000))
    in_topk = (vb > v_eff) | ((vb == v_eff) & (idx <= i_thr))

    sa = sa_ref[0, 0, 0]
    sb = sb_ref[0, 0, 0]
    noisy = sa * od_t + sb * noise
    resid = noise - noisy * w1
    tgt = om_t * jnp.where(in_topk | (rf == 0.0), 1.0, 0.0)
    num = jnp.sum(tgt * resid * resid)
    cnt = jnp.sum(tgt)
    loss_b = num / (cnt + 1e-6)

    @pl.when(b == 0)
    def _():
        out_ref[0, 0] = 0.0

    out_ref[0, 0] += loss_b * inv_b


def kernel(observed_data, observed_mask, timepoints, gt_mask, t, noise,
           rand_vals, sample_ratios, w1, w2):
    B, L, K = observed_data.shape
    ab = jnp.asarray(_alpha_bar_np(), jnp.float32)[t]
    sa = jnp.sqrt(ab).reshape(B, 1, 1)
    sb = jnp.sqrt(1.0 - ab).reshape(B, 1, 1)
    ratios = sample_ratios.reshape(B, 1, 1).astype(jnp.float32)
    tie_iters = max(1, int(np.ceil(np.log2(K * L))))

    smem = lambda: pl.BlockSpec((1, 1, 1), lambda b: (b, 0, 0),
                                memory_space=pltpu.SMEM)
    body = functools.partial(_loss_kernel, inv_b=float(1.0 / B),
                             tie_iters=tie_iters)
    out = pl.pallas_call(
        body,
        grid=(B,),
        in_specs=[
            smem(), smem(), smem(),
            pl.BlockSpec((1, L, K), lambda b: (b, 0, 0)),
            pl.BlockSpec((1, L, K), lambda b: (b, 0, 0)),
            pl.BlockSpec((1, K, L), lambda b: (b, 0, 0)),
            pl.BlockSpec((1, K, L), lambda b: (b, 0, 0)),
            pl.BlockSpec((K, L), lambda b: (0, 0)),
        ],
        out_specs=pl.BlockSpec((1, 1), lambda b: (0, 0),
                               memory_space=pltpu.SMEM),
        out_shape=jax.ShapeDtypeStruct((1, 1), jnp.float32),
    )(sa, sb, ratios, observed_data, observed_mask, noise, rand_vals, w1)
    return out[0, 0]
